# Initial kernel scaffold; baseline (speedup 1.0000x reference)
import jax
import jax.numpy as jnp
from jax.experimental import pallas as pl


def kernel(atom_features, atom_split, protSeq_features):
    raise NotImplementedError("write your pallas kernel here")


# speedup vs baseline: 10.4705x; 10.4705x over previous
import functools

import jax
import jax.numpy as jnp
from jax import lax
from jax.experimental import pallas as pl
from jax.experimental.pallas import tpu as pltpu
from jax.experimental.pallas import tpu_sc as plsc

N = 320000
B = 10000
D = 128
L = 16
NV = D // L
NC = 2
NS = 16
NW = NC * NS
SPW = 313
SPWP = 320
BP = NW * SPW
K = 256
NEG = -3.0e38

_mesh = plsc.VectorSubcoreMesh(core_axis_name="c", subcore_axis_name="s")


@functools.partial(
    pl.kernel,
    out_type=jax.ShapeDtypeStruct((NW * SPWP * D,), jnp.float32),
    mesh=_mesh,
    compiler_params=pltpu.CompilerParams(needs_layout_passes=False),
    scratch_types=[
        pltpu.VMEM((56,), jnp.int32),
        pltpu.VMEM((K, D), jnp.float32),
        pltpu.VMEM((K + L,), jnp.int32),
        pltpu.VMEM((SPWP, D), jnp.float32),
        pltpu.VMEM((SPWP * D,), jnp.float32),
        pltpu.SemaphoreType.DMA,
    ],
)
def _wg(atom_hbm, ids_hbm, prot_hbm, starts_hbm, out_hbm,
        starts_v, ablk, iblk, protb, outb, sem):
    w = lax.axis_index("s") * NC + lax.axis_index("c")
    s_lo = w * SPW
    pltpu.sync_copy(starts_hbm, starts_v)
    pltpu.sync_copy(prot_hbm.at[w], protb)

    zvec = jnp.zeros((L,), jnp.float32)

    def _zero(r, c):
        outb[pl.ds(r * L, L)] = zvec
        return c

    lax.fori_loop(0, SPWP * D // L, _zero, 0)

    svec = starts_v[pl.ds(w, L)]
    a0 = svec[0]
    a1 = svec[1]
    base = a0 - lax.rem(a0, 8)
    nblk = lax.div(a1 - base + (K - 1), K)

    def finalize(cur, s, acc):
        @pl.when(cur >= 0)
        def _():
            inv = 1.0 / (s + 1e-12)
            r = cur - s_lo
            for j in range(NV):
                outb[pl.ds(r * D + j * L, L)] = acc[j] * inv

    def blk_body(b, carry):
        off = pl.multiple_of(jnp.minimum(base + b * K, N - K), 8)
        pltpu.sync_copy(atom_hbm.at[pl.ds(off, K), :], ablk)
        pltpu.sync_copy(ids_hbm.at[pl.ds(off, K)], iblk.at[pl.ds(0, K)])
        lo = jnp.maximum(a0, base + b * K) - off
        hi = jnp.minimum(a1, base + (b + 1) * K) - off

        def atom_body(i, c):
            cur, m, s, acc = c
            sid = iblk[pl.ds(i, L)][0]
            is_new = sid != cur

            @pl.when(jnp.logical_and(is_new, cur >= 0))
            def _():
                finalize(cur, s, acc)

            m = jnp.where(is_new, jnp.full((L,), NEG, jnp.float32), m)
            s = jnp.where(is_new, zvec, s)
            acc = tuple(jnp.where(is_new, zvec, acc[j]) for j in range(NV))

            r = sid - s_lo
            a = tuple(ablk[i, pl.ds(j * L, L)] for j in range(NV))
            p = tuple(protb[r, pl.ds(j * L, L)] for j in range(NV))
            part = a[0] * p[0]
            for j in range(1, NV):
                part = part + a[j] * p[j]
            lv = jnp.full((L,), jnp.sum(part), jnp.float32)
            mn = jnp.maximum(m, lv)
            sc = jnp.exp(m - mn)
            e = jnp.exp(lv - mn)
            s = s * sc + e
            acc = tuple(acc[j] * sc + a[j] * e for j in range(NV))
            return (sid, mn, s, acc)

        return lax.fori_loop(lo, hi, atom_body, carry)

    init = (jnp.int32(-1), jnp.full((L,), NEG, jnp.float32), zvec,
            tuple(zvec for _ in range(NV)))
    cur, m, s, acc = lax.fori_loop(0, nblk, blk_body, init)
    finalize(cur, s, acc)
    pltpu.sync_copy(
        outb, out_hbm.at[pl.ds(pl.multiple_of(w * (SPWP * D), 8), SPWP * D)])


def kernel(atom_features, atom_split, protSeq_features):
    ids = atom_split.astype(jnp.int32)
    bounds = (jnp.arange(NW + 1, dtype=jnp.int32) * SPW)
    starts = jnp.searchsorted(ids, bounds).astype(jnp.int32)
    starts = jnp.concatenate(
        [starts, jnp.full((56 - (NW + 1),), N, jnp.int32)])
    protp = jnp.concatenate(
        [protSeq_features,
         jnp.zeros((BP - B, D), jnp.float32)]).reshape(NW, SPW, D)
    protp = jnp.pad(protp, ((0, 0), (0, SPWP - SPW), (0, 0)))
    out = _wg(atom_features, ids, protp, starts)
    return out.reshape(NW, SPWP, D)[:, :SPW, :].reshape(BP, D)[:B]
